# baseline (speedup 1.0000x reference)
import functools

import jax
import jax.numpy as jnp
from jax import lax
from jax.experimental import pallas as pl
from jax.experimental.pallas import tpu as pltpu
from jax.experimental.pallas import tpu_sc as plsc

BLK = 128
DEPTH = 6
NTR = 3


@functools.partial(jax.jit, static_argnums=(2, 3))
def _lookup(idx2, table, n_units, num_cores):
    V, D = table.shape
    S = 200
    NBT = 32
    per_w = n_units // 32
    mesh = plsc.VectorSubcoreMesh(core_axis_name="c", subcore_axis_name="s")

    @functools.partial(
        pl.kernel,
        mesh=mesh,
        compiler_params=pltpu.CompilerParams(
            use_tc_tiling_on_sc=False, needs_layout_passes=False),
        out_type=jax.ShapeDtypeStruct((S, 8, NBT, 8, BLK), jnp.float32),
        scratch_types=[
            pltpu.VMEM((per_w, BLK), jnp.int32),
            pltpu.VMEM((DEPTH, BLK, D), jnp.float32),
            pltpu.VMEM((NTR, 8, 2, 8, BLK + 1), jnp.float32),
            pltpu.SemaphoreType.DMA,
            pltpu.SemaphoreType.DMA,
        ],
    )
    def k(idx_hbm, table_hbm, out_hbm, idx_v, rows_v, tr_v, gsem, wsem):
        wid = lax.axis_index("s") * num_cores + lax.axis_index("c")
        u0 = wid * per_w
        pltpu.sync_copy(idx_hbm.at[pl.ds(u0, per_w)], idx_v)

        def g_start(j, b):
            pltpu.make_async_copy(
                table_hbm.at[idx_v.at[j]], rows_v.at[b], gsem).start()

        def g_wait(b):
            pltpu.make_async_copy(
                table_hbm.at[idx_v.at[0]], rows_v.at[b], gsem).wait()

        def w_start(jp, t):
            u = u0 + 2 * jp
            s = u // NBT
            bt = u % NBT
            pltpu.make_async_copy(
                tr_v.at[t, :, :, :, pl.ds(0, BLK)],
                out_hbm.at[s, :, pl.ds(bt, 2)], wsem).start()

        def w_wait(t):
            pltpu.make_async_copy(
                tr_v.at[t, :, :, :, pl.ds(0, BLK)],
                out_hbm.at[0, :, pl.ds(0, 2)], wsem).wait()

        def transpose(b, t, q):
            rv = rows_v.at[b]
            tv = tr_v.at[t]
            iota = jax.lax.broadcasted_iota(jnp.int32, (16,), 0)
            ets = [(iota + 16 * ge) >> 3 for ge in range(D // 16)]
            eis = [(iota + 16 * ge) & 7 for ge in range(D // 16)]
            qv = (iota & 0) + q

            def brow(j):
                col = jnp.full((16,), j, jnp.int32)
                vs = [rv[j, pl.ds(16 * ge, 16)] for ge in range(D // 16)]
                for ge in range(D // 16):
                    plsc.store_scatter(
                        tv, [ets[ge], qv, eis[ge], col], vs[ge])

            plsc.parallel_loop(0, BLK, 1, unroll=4)(brow)

        for b in range(DEPTH):
            g_start(b, b)

        n_pairs = per_w // 2

        def pair(jp, carry):
            t = jp % NTR

            @pl.when(jp >= NTR)
            def _():
                w_wait(t)

            for q in range(2):
                j = 2 * jp + q
                b = j % DEPTH
                g_wait(b)
                transpose(b, t, q)

                @pl.when(j + DEPTH < per_w)
                def _():
                    g_start(j + DEPTH, b)

            w_start(jp, t)
            return carry

        lax.fori_loop(0, n_pairs, pair, 0)
        for t in range(NTR):
            w_wait(t)

    return k(idx2, table)


def kernel(batch, word_embeddings):
    B0, B1 = batch.shape
    D = word_embeddings.shape[1]
    info = plsc.get_sparse_core_info()
    n_units = B1 * (B0 // BLK)
    idx2 = batch.astype(jnp.int32).T.reshape(n_units, BLK)
    out5 = _lookup(idx2, word_embeddings, n_units, info.num_cores)
    out3 = out5.transpose(0, 1, 3, 2, 4).reshape(B1, D, B0)
    return out3.transpose(2, 0, 1)


# speedup vs baseline: 1.0208x; 1.0208x over previous
import functools

import jax
import jax.numpy as jnp
from jax import lax
from jax.experimental import pallas as pl
from jax.experimental.pallas import tpu as pltpu
from jax.experimental.pallas import tpu_sc as plsc

BLK = 128
DEPTH = 7
NTR = 5


@functools.partial(jax.jit, static_argnums=(2, 3))
def _lookup(idx2, table, n_units, num_cores):
    V, D = table.shape
    S = 200
    NBT = 32
    per_w = n_units // 32
    mesh = plsc.VectorSubcoreMesh(core_axis_name="c", subcore_axis_name="s")

    @functools.partial(
        pl.kernel,
        mesh=mesh,
        compiler_params=pltpu.CompilerParams(
            use_tc_tiling_on_sc=False, needs_layout_passes=False),
        out_type=jax.ShapeDtypeStruct((S, 8, NBT, 8, BLK), jnp.float32),
        scratch_types=[
            pltpu.VMEM((per_w, BLK), jnp.int32),
            pltpu.VMEM((DEPTH, BLK, D), jnp.float32),
            pltpu.VMEM((NTR, 8, 8, BLK + 1), jnp.float32),
            pltpu.SemaphoreType.DMA,
            pltpu.SemaphoreType.DMA,
        ],
    )
    def k(idx_hbm, table_hbm, out_hbm, idx_v, rows_v, tr_v, gsem, wsem):
        wid = lax.axis_index("s") * num_cores + lax.axis_index("c")
        u0 = wid * per_w
        pltpu.sync_copy(idx_hbm.at[pl.ds(u0, per_w)], idx_v)

        def g_start(j, b):
            pltpu.make_async_copy(
                table_hbm.at[idx_v.at[j]], rows_v.at[b], gsem).start()

        def g_wait(b):
            pltpu.make_async_copy(
                table_hbm.at[idx_v.at[0]], rows_v.at[b], gsem).wait()

        def w_start(j, t):
            u = u0 + j
            s = u // NBT
            bt = u % NBT
            pltpu.make_async_copy(
                tr_v.at[t, :, :, pl.ds(0, BLK)], out_hbm.at[s, :, bt],
                wsem).start()

        def w_wait(t):
            pltpu.make_async_copy(
                tr_v.at[t, :, :, pl.ds(0, BLK)], out_hbm.at[0, :, 0],
                wsem).wait()

        def transpose(b, t):
            rv = rows_v.at[b]
            tv = tr_v.at[t]
            iota = jax.lax.broadcasted_iota(jnp.int32, (16,), 0)
            ets = [(iota + 16 * ge) >> 3 for ge in range(D // 16)]
            eis = [(iota + 16 * ge) & 7 for ge in range(D // 16)]

            def brow(j):
                col = jnp.full((16,), j, jnp.int32)
                vs = [rv[j, pl.ds(16 * ge, 16)] for ge in range(D // 16)]
                for ge in range(D // 16):
                    plsc.store_scatter(tv, [ets[ge], eis[ge], col], vs[ge])

            plsc.parallel_loop(0, BLK, 1, unroll=4)(brow)

        for b in range(DEPTH):
            g_start(b, b)

        def unit(j, carry):
            b = j % DEPTH
            t = j % NTR
            g_wait(b)

            @pl.when(j >= NTR)
            def _():
                w_wait(t)

            transpose(b, t)

            @pl.when(j + DEPTH < per_w)
            def _():
                g_start(j + DEPTH, b)

            w_start(j, t)
            return carry

        lax.fori_loop(0, per_w, unit, 0)
        for t in range(NTR):
            w_wait(t)

    return k(idx2, table)


def kernel(batch, word_embeddings):
    B0, B1 = batch.shape
    D = word_embeddings.shape[1]
    info = plsc.get_sparse_core_info()
    n_units = B1 * (B0 // BLK)
    idx2 = batch.astype(jnp.int32).T.reshape(n_units, BLK)
    out5 = _lookup(idx2, word_embeddings, n_units, info.num_cores)
    out3 = out5.transpose(0, 1, 3, 2, 4).reshape(B1, D, B0)
    return out3.transpose(2, 0, 1)
